# baseline (speedup 1.0000x reference)
import functools

import jax
import jax.numpy as jnp
from jax import lax
from jax.experimental import pallas as pl
from jax.experimental.pallas import tpu as pltpu
from jax.experimental.pallas import tpu_sc as plsc

B = 16384
H = 200
E = 32
HID = 128
OUT = 2

NC, NS = 2, 16
NW = NC * NS
RPG = 128
G = B // RPG
GPW = G // NW
IPG = RPG * H

_mesh = plsc.VectorSubcoreMesh(
    core_axis_name="c", subcore_axis_name="s", num_cores=NC, num_subcores=NS
)


HC = H // 16
HREM = H - HC * 16


@functools.partial(
    pl.kernel,
    out_type=jax.ShapeDtypeStruct((B, E), jnp.float32),
    mesh=_mesh,
    scratch_types=[
        pltpu.VMEM((IPG + 16,), jnp.int32),
        pltpu.VMEM((IPG,), jnp.int32),
        pltpu.VMEM((RPG, E), jnp.float32),
        pltpu.SemaphoreType.DMA,
    ],
    compiler_params=pltpu.CompilerParams(
        use_tc_tiling_on_sc=False, needs_layout_passes=False
    ),
)
def _sc_pool(xg, table, out, xrow_v, idx_v, acc, sem):
    wid = lax.axis_index("s") * NC + lax.axis_index("c")
    zero = jnp.zeros((16,), jnp.float32)
    ivec = jnp.arange(16, dtype=jnp.int32)
    ivec128 = ivec * RPG
    tail_mask = ivec < HREM
    for g in range(GPW):
        gg = wid * GPW + g
        pltpu.sync_copy(xg.at[pl.ds(gg * IPG, IPG)], xrow_v.at[pl.ds(0, IPG)])

        def tr_body(r, carry):
            for hc in range(HC):
                v = xrow_v[pl.ds(r * H + hc * 16, 16)]
                plsc.store_scatter(idx_v, [ivec128 + (hc * 16 * RPG + r)], v)
            v = xrow_v[pl.ds(r * H + HC * 16, 16)]
            plsc.store_scatter(
                idx_v, [ivec128 + (HC * 16 * RPG + r)], v, mask=tail_mask
            )
            return carry

        lax.fori_loop(0, RPG, tr_body, 0)

        def zero_body(r, carry):
            acc[r, pl.ds(0, 16)] = zero
            acc[r, pl.ds(16, 16)] = zero
            return carry

        lax.fori_loop(0, RPG, zero_body, 0, unroll=8)

        def fire_body(h, carry):
            pltpu.async_copy(
                table.at[idx_v.at[pl.ds(h * RPG, RPG)]], acc, sem, add=True
            )
            return carry

        lax.fori_loop(0, H, fire_body, 0)

        def drain_body(h, carry):
            pltpu.make_async_copy(table.at[idx_v.at[pl.ds(0, RPG)]], acc, sem).wait()
            return carry

        lax.fori_loop(0, H, drain_body, 0)

        pltpu.sync_copy(acc, out.at[pl.ds(gg * RPG, RPG)])


def _mlp_body(p_ref, w1_ref, b1_ref, w2_ref, b2_ref, o_ref):
    h = jnp.dot(p_ref[...], w1_ref[...], preferred_element_type=jnp.float32)
    h = h + b1_ref[...]
    o = jnp.dot(h, w2_ref[...], preferred_element_type=jnp.float32)
    o_ref[...] = o + b2_ref[...]


_MLP_BLOCK = 2048
_mlp = pl.pallas_call(
    _mlp_body,
    grid=(B // _MLP_BLOCK,),
    in_specs=[
        pl.BlockSpec((_MLP_BLOCK, E), lambda i: (i, 0)),
        pl.BlockSpec((E, HID), lambda i: (0, 0)),
        pl.BlockSpec((1, HID), lambda i: (0, 0)),
        pl.BlockSpec((HID, OUT), lambda i: (0, 0)),
        pl.BlockSpec((1, OUT), lambda i: (0, 0)),
    ],
    out_specs=pl.BlockSpec((_MLP_BLOCK, OUT), lambda i: (i, 0)),
    out_shape=jax.ShapeDtypeStruct((B, OUT), jnp.float32),
)


@jax.jit
def kernel(x, embeddings, W1, b1, W2, b2):
    xg = x.astype(jnp.int32).reshape(-1)
    pooled = _sc_pool(xg, embeddings)
    return _mlp(pooled, W1, b1.reshape(1, HID), W2, b2.reshape(1, OUT))
